# baseline (device time: 22200 ns/iter reference)
import jax
import jax.numpy as jnp
from jax import lax
from jax.experimental import pallas as pl
from jax.experimental.pallas import tpu as pltpu

K = 16
NEG_INF = float("-inf")
N_BLK = 2


def _topk_cols(xv, k):
    cols = []
    for _ in range(k):
        m = jnp.max(xv, axis=1, keepdims=True)
        cols.append(m)
        xv = jnp.where(xv == m, NEG_INF, xv)
    return cols


def _block_topk(xv):
    rows, n = xv.shape
    x3 = xv.reshape(rows, n // 128, 128)
    cands = []
    for _ in range(4):
        li = jnp.max(x3, axis=1)
        cands.append(li)
        x3 = jnp.where(x3 == li[:, None, :], NEG_INF, x3)
    cols = _topk_cols(jnp.concatenate(cands, axis=1), K)
    return jnp.concatenate(cols, axis=1), jnp.concatenate(cols[::-1], axis=1)


def _bitonic_desc(u):
    rows = u.shape[0]
    for d in (8, 4, 2, 1):
        g = u.reshape(rows, 16 // (2 * d), 2, d)
        hi = jnp.maximum(g[:, :, 0, :], g[:, :, 1, :])
        lo = jnp.minimum(g[:, :, 0, :], g[:, :, 1, :])
        u = jnp.stack([hi, lo], axis=2).reshape(rows, 16)
    return u


def _merge16(a_desc, b_asc):
    return _bitonic_desc(jnp.maximum(a_desc, b_asc))


def kernel(x):
    rows, n = x.shape
    half = rows // 2
    blk = half // N_BLK

    def body(x_ref, out_ref, loc_ref, asc_ref, rx_ref, ry_ref, rd_ref,
             sx_sems, rx_sems, sy_sems, ry_sems, sd_sems, rd_sems):
        my_x = lax.axis_index("x")
        my_y = lax.axis_index("y")
        nbr_x = (1 - my_x, my_y)
        nbr_y = (my_x, 1 - my_y)
        nbr_d = (1 - my_x, 1 - my_y)
        row0 = my_y * half
        orow0 = (1 - my_y) * half

        barrier_sem = pltpu.get_barrier_semaphore()
        for nbr in (nbr_x, nbr_y, nbr_d):
            pl.semaphore_signal(
                barrier_sem, inc=1, device_id=nbr,
                device_id_type=pl.DeviceIdType.MESH,
            )
        pl.semaphore_wait(barrier_sem, 3)

        def rdma(b, src, dst, ssem, rsem, nbr):
            return pltpu.make_async_remote_copy(
                src_ref=src.at[b],
                dst_ref=dst.at[b],
                send_sem=ssem.at[b],
                recv_sem=rsem.at[b],
                device_id=nbr,
                device_id_type=pl.DeviceIdType.MESH,
            )

        rdma_x = lambda b: rdma(b, asc_ref, rx_ref, sx_sems, rx_sems, nbr_x)
        rdma_y = lambda b: rdma(b, loc_ref, ry_ref, sy_sems, ry_sems, nbr_y)
        rdma_d = lambda b: rdma(b, asc_ref, rd_ref, sd_sems, rd_sems, nbr_d)

        for b in range(N_BLK):
            desc, asc = _block_topk(x_ref[pl.ds(row0 + b * blk, blk), :])
            loc_ref[b] = desc
            asc_ref[b] = asc
            rdma_x(b).start()
            rdma_y(b).start()
            rdma_d(b).start()

        for b in range(N_BLK):
            rdma_x(b).wait()
            out_ref[pl.ds(row0 + b * blk, blk), :] = _merge16(
                loc_ref[b], rx_ref[b]
            )
        for b in range(N_BLK):
            rdma_y(b).wait()
            rdma_d(b).wait()
            out_ref[pl.ds(orow0 + b * blk, blk), :] = _merge16(
                ry_ref[b], rd_ref[b]
            )

    return pl.pallas_call(
        body,
        out_shape=jax.ShapeDtypeStruct((rows, K), jnp.float32),
        in_specs=[pl.BlockSpec(memory_space=pltpu.VMEM)],
        out_specs=pl.BlockSpec(memory_space=pltpu.VMEM),
        scratch_shapes=[
            pltpu.VMEM((N_BLK, blk, K), jnp.float32),
            pltpu.VMEM((N_BLK, blk, K), jnp.float32),
            pltpu.VMEM((N_BLK, blk, K), jnp.float32),
            pltpu.VMEM((N_BLK, blk, K), jnp.float32),
            pltpu.VMEM((N_BLK, blk, K), jnp.float32),
            pltpu.SemaphoreType.DMA((N_BLK,)),
            pltpu.SemaphoreType.DMA((N_BLK,)),
            pltpu.SemaphoreType.DMA((N_BLK,)),
            pltpu.SemaphoreType.DMA((N_BLK,)),
            pltpu.SemaphoreType.DMA((N_BLK,)),
            pltpu.SemaphoreType.DMA((N_BLK,)),
        ],
        compiler_params=pltpu.CompilerParams(collective_id=0),
    )(x)
